# Initial kernel scaffold; baseline (speedup 1.0000x reference)
import jax
import jax.numpy as jnp
from jax.experimental import pallas as pl


def kernel(z, keys, source_sink, W, b):
    raise NotImplementedError("write your pallas kernel here")


# speedup vs baseline: 8.5425x; 8.5425x over previous
import dataclasses
import functools

import jax
import jax.numpy as jnp
from jax import lax
from jax.experimental import pallas as pl
from jax.experimental.pallas import tpu as pltpu
from jax.experimental.pallas import tpu_sc as plsc

N_NODES = 10000
N_EDGES = 320000
LATENT = 128
INPUT = 128

NC = 2
NS = 16
NW = NC * NS
EPW = N_EDGES // NW
K = 80
NB = EPW // K
N_PAD = 10240
RPW = N_PAD // NS
def _sc_segment_sum(z, sources, sinks, zrow_zeros, cnt_zeros):
    mesh = plsc.VectorSubcoreMesh(
        core_axis_name="c", subcore_axis_name="s",
        num_cores=NC, num_subcores=NS)

    cp = pltpu.CompilerParams()
    if "needs_layout_passes" in pltpu.CompilerParams.__dataclass_fields__:
        cp = dataclasses.replace(cp, needs_layout_passes=False)

    @functools.partial(
        pl.kernel,
        out_type=[
            jax.ShapeDtypeStruct((NC, N_PAD, LATENT), jnp.float32),
            jax.ShapeDtypeStruct((NC, NS, N_PAD), jnp.float32),
        ],
        compiler_params=cp,
        mesh=mesh,
        scratch_types=[
            pltpu.VMEM_SHARED((N_PAD, LATENT), jnp.float32),
            pltpu.VMEM((1, K), jnp.int32),
            pltpu.VMEM((1, K), jnp.int32),
            pltpu.VMEM((1, K), jnp.int32),
            pltpu.VMEM((K, LATENT), jnp.float32),
            pltpu.VMEM((N_PAD,), jnp.float32),
            pltpu.SemaphoreType.DMA,
        ],
    )
    def sc_kernel(z_hbm, src_hbm, snk_hbm, ar_hbm, zz_hbm, cz_hbm,
                  a_out, c_out, acc, sidx_v, tidx_v, iidx_v,
                  rows_v, cnt_v, sem):
        c = lax.axis_index("c")
        s = lax.axis_index("s")
        wid = c * NS + s
        pltpu.sync_copy(zz_hbm, rows_v)
        pltpu.sync_copy(cz_hbm, cnt_v)

        @pl.loop(0, RPW // K)
        def _(j):
            pltpu.sync_copy(ar_hbm.at[pl.ds(s * RPW + j * K, K)],
                            iidx_v.at[0])
            pltpu.sync_copy(rows_v, acc.at[iidx_v.at[0]])

        plsc.subcore_barrier()
        ones16 = jnp.ones((16,), jnp.float32)

        @pl.loop(0, NB)
        def _(i):
            base = wid * EPW + i * K
            pltpu.sync_copy(src_hbm.at[pl.ds(base, K)], sidx_v.at[0])
            pltpu.sync_copy(snk_hbm.at[pl.ds(base, K)], tidx_v.at[0])
            pltpu.async_copy(z_hbm.at[tidx_v.at[0]], rows_v, sem).wait()
            pltpu.sync_copy(rows_v, acc.at[sidx_v.at[0]], add=True)

            @pl.loop(0, K // 16)
            def _(jj):
                idx16 = sidx_v[0, pl.ds(jj * 16, 16)]
                plsc.addupdate_scatter(cnt_v, [idx16], ones16)

        plsc.subcore_barrier()

        @pl.loop(0, RPW // K)
        def _(j):
            r = s * RPW + j * K
            pltpu.sync_copy(ar_hbm.at[pl.ds(r, K)], iidx_v.at[0])
            pltpu.async_copy(acc.at[iidx_v.at[0]], rows_v, sem).wait()
            pltpu.sync_copy(rows_v, a_out.at[c, pl.ds(r, K)])

        pltpu.sync_copy(cnt_v, c_out.at[c, s])

    return sc_kernel(z, sources, sinks,
                     jnp.arange(N_PAD, dtype=jnp.int32),
                     zrow_zeros, cnt_zeros)


def _tc_combine(keys, a0, a1, cw, w1t, w2t, b2):
    BLK = 1024

    def body(keys_b, a0_b, a1_b, cw_b, w1_b, w2_b, b_b, o_b):
        cnt = jnp.sum(cw_b[...], axis=0)[:, None]
        kk = jnp.dot(keys_b[...], w1_b[...],
                     preferred_element_type=jnp.float32)
        aa = jnp.dot(a0_b[...] + a1_b[...], w2_b[...],
                     preferred_element_type=jnp.float32)
        o_b[...] = (cnt * (kk + b_b[...]) + aa) / jnp.maximum(cnt, 1.0)

    row_spec = pl.BlockSpec((BLK, LATENT), lambda i: (i, 0))
    cnt_spec = pl.BlockSpec((NW, BLK), lambda i: (0, i))
    mat_spec = pl.BlockSpec((LATENT, LATENT), lambda i: (0, 0))
    return pl.pallas_call(
        body,
        grid=((N_NODES + BLK - 1) // BLK,),
        in_specs=[row_spec, row_spec, row_spec, cnt_spec,
                  mat_spec, mat_spec, pl.BlockSpec((1, INPUT), lambda i: (0, 0))],
        out_specs=pl.BlockSpec((BLK, INPUT), lambda i: (i, 0)),
        out_shape=jax.ShapeDtypeStruct((N_NODES, INPUT), jnp.float32),
    )(keys, a0, a1, cw, w1t, w2t, b2)


def kernel(z, keys, source_sink, W, b):
    zrow_zeros = jnp.zeros((K, LATENT), jnp.float32)
    cnt_zeros = jnp.zeros((N_PAD,), jnp.float32)
    a_part, c_part = _sc_segment_sum(z, source_sink[0], source_sink[1],
                                     zrow_zeros, cnt_zeros)
    w1t = W[:, :LATENT].T
    w2t = W[:, LATENT:].T
    b2 = b.reshape(1, INPUT)
    return _tc_combine(keys, a_part[0], a_part[1],
                       c_part.reshape(NW, N_PAD), w1t, w2t, b2)


# speedup vs baseline: 14.5225x; 1.7000x over previous
import dataclasses
import functools

import jax
import jax.numpy as jnp
from jax import lax
from jax.experimental import pallas as pl
from jax.experimental.pallas import tpu as pltpu
from jax.experimental.pallas import tpu_sc as plsc

N_NODES = 10000
N_EDGES = 320000
LATENT = 128
INPUT = 128

NC = 2
NS = 16
NW = NC * NS
EPW = N_EDGES // NW
K = 80
NB = EPW // K
N_PAD = 10240
RPW = N_PAD // NS
def _sc_segment_sum(z, sources, sinks, zrow_zeros, cnt_zeros):
    mesh = plsc.VectorSubcoreMesh(
        core_axis_name="c", subcore_axis_name="s",
        num_cores=NC, num_subcores=NS)

    cp = pltpu.CompilerParams()
    if "needs_layout_passes" in pltpu.CompilerParams.__dataclass_fields__:
        cp = dataclasses.replace(cp, needs_layout_passes=False)

    @functools.partial(
        pl.kernel,
        out_type=[
            jax.ShapeDtypeStruct((NC, N_PAD, LATENT), jnp.float32),
            jax.ShapeDtypeStruct((NC, NS, N_PAD), jnp.float32),
        ],
        compiler_params=cp,
        mesh=mesh,
        scratch_types=[
            pltpu.VMEM_SHARED((N_PAD, LATENT), jnp.float32),
            pltpu.VMEM((1, K), jnp.int32),
            pltpu.VMEM((1, K), jnp.int32),
            pltpu.VMEM((1, K), jnp.int32),
            pltpu.VMEM((1, K), jnp.int32),
            pltpu.VMEM((1, K), jnp.int32),
            pltpu.VMEM((K, LATENT), jnp.float32),
            pltpu.VMEM((K, LATENT), jnp.float32),
            pltpu.VMEM((N_PAD,), jnp.float32),
            pltpu.SemaphoreType.DMA,
            pltpu.SemaphoreType.DMA,
            pltpu.SemaphoreType.DMA,
            pltpu.SemaphoreType.DMA,
            pltpu.SemaphoreType.DMA,
            pltpu.SemaphoreType.DMA,
            pltpu.SemaphoreType.DMA,
        ],
    )
    def sc_kernel(z_hbm, src_hbm, snk_hbm, ar_hbm, zz_hbm, cz_hbm,
                  a_out, c_out, acc, sidx_a, tidx_a, sidx_b, tidx_b, iidx_v,
                  rows_a, rows_b, cnt_v,
                  sem, is_a, is_b, gs_a, gs_b, ss_a, ss_b):
        c = lax.axis_index("c")
        s = lax.axis_index("s")
        wid = c * NS + s
        pltpu.sync_copy(zz_hbm, rows_a)
        pltpu.sync_copy(cz_hbm, cnt_v)

        @pl.loop(0, RPW // K)
        def _(j):
            pltpu.sync_copy(ar_hbm.at[pl.ds(s * RPW + j * K, K)],
                            iidx_v.at[0])
            pltpu.sync_copy(rows_a, acc.at[iidx_v.at[0]])

        plsc.subcore_barrier()
        ones16 = jnp.ones((16,), jnp.float32)

        def load_idx(i, sidx, tidx, isem):
            base = wid * EPW + i * K
            pltpu.async_copy(src_hbm.at[pl.ds(base, K)], sidx.at[0], isem)
            pltpu.async_copy(snk_hbm.at[pl.ds(base, K)], tidx.at[0], isem)

        def wait_idx(sidx, tidx, isem):
            pltpu.make_async_copy(src_hbm.at[pl.ds(0, K)], sidx.at[0],
                                  isem).wait()
            pltpu.make_async_copy(snk_hbm.at[pl.ds(0, K)], tidx.at[0],
                                  isem).wait()

        def issue_gather(tidx, rows, gsem):
            pltpu.async_copy(z_hbm.at[tidx.at[0]], rows, gsem)

        def wait_gather(tidx, rows, gsem):
            pltpu.make_async_copy(z_hbm.at[tidx.at[0]], rows, gsem).wait()

        def issue_scat(rows, sidx, ssem):
            pltpu.async_copy(rows, acc.at[sidx.at[0]], ssem, add=True)

        def wait_scat(rows, sidx, ssem):
            pltpu.make_async_copy(rows, acc.at[sidx.at[0]], ssem).wait()

        def counts(sidx):
            @pl.loop(0, K // 16)
            def _(jj):
                idx16 = sidx[0, pl.ds(jj * 16, 16)]
                plsc.addupdate_scatter(cnt_v, [idx16], ones16)

        load_idx(0, sidx_a, tidx_a, is_a)
        load_idx(1, sidx_b, tidx_b, is_b)
        wait_idx(sidx_a, tidx_a, is_a)
        issue_gather(tidx_a, rows_a, gs_a)

        @pl.loop(0, (NB - 1) // 2)
        def _(j):
            e = 2 * j
            wait_gather(tidx_a, rows_a, gs_a)
            issue_scat(rows_a, sidx_a, ss_a)
            counts(sidx_a)
            wait_idx(sidx_b, tidx_b, is_b)
            issue_gather(tidx_b, rows_b, gs_b)
            wait_scat(rows_a, sidx_a, ss_a)
            load_idx(e + 2, sidx_a, tidx_a, is_a)
            wait_gather(tidx_b, rows_b, gs_b)
            issue_scat(rows_b, sidx_b, ss_b)
            counts(sidx_b)
            wait_idx(sidx_a, tidx_a, is_a)
            issue_gather(tidx_a, rows_a, gs_a)
            wait_scat(rows_b, sidx_b, ss_b)

            @pl.when(e + 3 < NB)
            def _():
                load_idx(e + 3, sidx_b, tidx_b, is_b)

        wait_gather(tidx_a, rows_a, gs_a)
        issue_scat(rows_a, sidx_a, ss_a)
        counts(sidx_a)
        wait_scat(rows_a, sidx_a, ss_a)

        plsc.subcore_barrier()

        @pl.loop(0, RPW // K)
        def _(j):
            r = s * RPW + j * K
            pltpu.sync_copy(ar_hbm.at[pl.ds(r, K)], iidx_v.at[0])
            pltpu.async_copy(acc.at[iidx_v.at[0]], rows_a, sem).wait()
            pltpu.sync_copy(rows_a, a_out.at[c, pl.ds(r, K)])

        pltpu.sync_copy(cnt_v, c_out.at[c, s])

    return sc_kernel(z, sources, sinks,
                     jnp.arange(N_PAD, dtype=jnp.int32),
                     zrow_zeros, cnt_zeros)


def _tc_combine(keys, a0, a1, cw, w1t, w2t, b2):
    BLK = 1024

    def body(keys_b, a0_b, a1_b, cw_b, w1_b, w2_b, b_b, o_b):
        cnt = jnp.sum(cw_b[...], axis=0)[:, None]
        kk = jnp.dot(keys_b[...], w1_b[...],
                     preferred_element_type=jnp.float32)
        aa = jnp.dot(a0_b[...] + a1_b[...], w2_b[...],
                     preferred_element_type=jnp.float32)
        o_b[...] = (cnt * (kk + b_b[...]) + aa) / jnp.maximum(cnt, 1.0)

    row_spec = pl.BlockSpec((BLK, LATENT), lambda i: (i, 0))
    cnt_spec = pl.BlockSpec((NW, BLK), lambda i: (0, i))
    mat_spec = pl.BlockSpec((LATENT, LATENT), lambda i: (0, 0))
    return pl.pallas_call(
        body,
        grid=((N_NODES + BLK - 1) // BLK,),
        in_specs=[row_spec, row_spec, row_spec, cnt_spec,
                  mat_spec, mat_spec, pl.BlockSpec((1, INPUT), lambda i: (0, 0))],
        out_specs=pl.BlockSpec((BLK, INPUT), lambda i: (i, 0)),
        out_shape=jax.ShapeDtypeStruct((N_NODES, INPUT), jnp.float32),
    )(keys, a0, a1, cw, w1t, w2t, b2)


def kernel(z, keys, source_sink, W, b):
    zrow_zeros = jnp.zeros((K, LATENT), jnp.float32)
    cnt_zeros = jnp.zeros((N_PAD,), jnp.float32)
    a_part, c_part = _sc_segment_sum(z, source_sink[0], source_sink[1],
                                     zrow_zeros, cnt_zeros)
    w1t = W[:, :LATENT].T
    w2t = W[:, LATENT:].T
    b2 = b.reshape(1, INPUT)
    return _tc_combine(keys, a_part[0], a_part[1],
                       c_part.reshape(NW, N_PAD), w1t, w2t, b2)


# speedup vs baseline: 15.2232x; 1.0482x over previous
import dataclasses
import functools

import jax
import jax.numpy as jnp
from jax import lax
from jax.experimental import pallas as pl
from jax.experimental.pallas import tpu as pltpu
from jax.experimental.pallas import tpu_sc as plsc

N_NODES = 10000
N_EDGES = 320000
LATENT = 128
INPUT = 128

NC = 2
NS = 16
NW = NC * NS
EPW = N_EDGES // NW
K = 80
NB = EPW // K
N_PAD = 10240
RPW = N_PAD // NS
def _sc_segment_sum(z, sources, sinks, zrow_zeros, cnt_zeros):
    mesh = plsc.VectorSubcoreMesh(
        core_axis_name="c", subcore_axis_name="s",
        num_cores=NC, num_subcores=NS)

    cp = pltpu.CompilerParams()
    if "needs_layout_passes" in pltpu.CompilerParams.__dataclass_fields__:
        cp = dataclasses.replace(cp, needs_layout_passes=False)

    @functools.partial(
        pl.kernel,
        out_type=[
            jax.ShapeDtypeStruct((NC, N_PAD, LATENT), jnp.float32),
            jax.ShapeDtypeStruct((NC, NS, N_PAD), jnp.float32),
        ],
        compiler_params=cp,
        mesh=mesh,
        scratch_types=[
            pltpu.VMEM_SHARED((N_PAD, LATENT), jnp.float32),
            pltpu.VMEM((1, K), jnp.int32),
            pltpu.VMEM((1, K), jnp.int32),
            pltpu.VMEM((1, K), jnp.int32),
            pltpu.VMEM((1, K), jnp.int32),
            pltpu.VMEM((RPW // K, K), jnp.int32),
            pltpu.VMEM((K, LATENT), jnp.float32),
            pltpu.VMEM((K, LATENT), jnp.float32),
            pltpu.VMEM((N_PAD,), jnp.float32),
            pltpu.SemaphoreType.DMA,
            pltpu.SemaphoreType.DMA,
            pltpu.SemaphoreType.DMA,
            pltpu.SemaphoreType.DMA,
            pltpu.SemaphoreType.DMA,
            pltpu.SemaphoreType.DMA,
            pltpu.SemaphoreType.DMA,
        ],
    )
    def sc_kernel(z_hbm, src_hbm, snk_hbm, ar_hbm, zz_hbm, cz_hbm,
                  a_out, c_out, acc, sidx_a, tidx_a, sidx_b, tidx_b, iidx8,
                  rows_a, rows_b, cnt_v,
                  sem, is_a, is_b, gs_a, gs_b, ss_a, ss_b):
        c = lax.axis_index("c")
        s = lax.axis_index("s")
        wid = c * NS + s
        NI = RPW // K
        ones16 = jnp.ones((16,), jnp.float32)

        def load_idx(i, sidx, tidx, isem):
            base = wid * EPW + i * K
            pltpu.async_copy(src_hbm.at[pl.ds(base, K)], sidx.at[0], isem)
            pltpu.async_copy(snk_hbm.at[pl.ds(base, K)], tidx.at[0], isem)

        def wait_idx(sidx, tidx, isem):
            pltpu.make_async_copy(src_hbm.at[pl.ds(0, K)], sidx.at[0],
                                  isem).wait()
            pltpu.make_async_copy(snk_hbm.at[pl.ds(0, K)], tidx.at[0],
                                  isem).wait()

        def issue_gather(tidx, rows, gsem):
            pltpu.async_copy(z_hbm.at[tidx.at[0]], rows, gsem)

        def wait_gather(tidx, rows, gsem):
            pltpu.make_async_copy(z_hbm.at[tidx.at[0]], rows, gsem).wait()

        def issue_scat(rows, sidx, ssem):
            pltpu.async_copy(rows, acc.at[sidx.at[0]], ssem, add=True)

        def wait_scat(rows, sidx, ssem):
            pltpu.make_async_copy(rows, acc.at[sidx.at[0]], ssem).wait()

        def counts(sidx):
            @pl.loop(0, K // 16)
            def _(jj):
                idx16 = sidx[0, pl.ds(jj * 16, 16)]
                plsc.addupdate_scatter(cnt_v, [idx16], ones16)

        pltpu.sync_copy(zz_hbm, rows_a)
        pltpu.sync_copy(cz_hbm, cnt_v)
        pltpu.sync_copy(ar_hbm.at[pl.ds(s * NI, NI)], iidx8)

        @pl.loop(0, NI)
        def _(j):
            pltpu.async_copy(rows_a, acc.at[iidx8.at[j]], sem)

        load_idx(0, sidx_a, tidx_a, is_a)
        load_idx(1, sidx_b, tidx_b, is_b)

        @pl.loop(0, NI)
        def _(j):
            pltpu.make_async_copy(rows_a, acc.at[iidx8.at[0]], sem).wait()

        plsc.subcore_barrier()

        wait_idx(sidx_a, tidx_a, is_a)
        issue_gather(tidx_a, rows_a, gs_a)

        @pl.loop(0, (NB - 1) // 2)
        def _(j):
            e = 2 * j
            wait_gather(tidx_a, rows_a, gs_a)
            issue_scat(rows_a, sidx_a, ss_a)
            counts(sidx_a)
            wait_idx(sidx_b, tidx_b, is_b)
            issue_gather(tidx_b, rows_b, gs_b)
            wait_scat(rows_a, sidx_a, ss_a)
            load_idx(e + 2, sidx_a, tidx_a, is_a)
            wait_gather(tidx_b, rows_b, gs_b)
            issue_scat(rows_b, sidx_b, ss_b)
            counts(sidx_b)
            wait_idx(sidx_a, tidx_a, is_a)
            issue_gather(tidx_a, rows_a, gs_a)
            wait_scat(rows_b, sidx_b, ss_b)

            @pl.when(e + 3 < NB)
            def _():
                load_idx(e + 3, sidx_b, tidx_b, is_b)

        wait_gather(tidx_a, rows_a, gs_a)
        issue_scat(rows_a, sidx_a, ss_a)
        counts(sidx_a)
        wait_scat(rows_a, sidx_a, ss_a)

        plsc.subcore_barrier()

        pltpu.async_copy(cnt_v, c_out.at[c, s], sem)

        def spg(j, rows, gsem):
            pltpu.async_copy(acc.at[iidx8.at[j]], rows, gsem)

        def wait_spg(rows, gsem):
            pltpu.make_async_copy(acc.at[iidx8.at[0]], rows, gsem).wait()

        def wout(j, rows, wsem):
            pltpu.async_copy(rows, a_out.at[c, pl.ds(s * RPW + j * K, K)],
                             wsem)

        def wait_wout(rows, wsem):
            pltpu.make_async_copy(rows, a_out.at[c, pl.ds(0, K)],
                                  wsem).wait()

        spg(0, rows_a, gs_a)

        @pl.loop(0, NI // 2)
        def _(j):
            e = 2 * j
            wait_spg(rows_a, gs_a)
            wout(e, rows_a, ss_a)
            spg(e + 1, rows_b, gs_b)
            wait_spg(rows_b, gs_b)
            wout(e + 1, rows_b, ss_b)
            wait_wout(rows_a, ss_a)

            @pl.when(e + 2 < NI)
            def _():
                spg(e + 2, rows_a, gs_a)

            wait_wout(rows_b, ss_b)

        pltpu.make_async_copy(cnt_v, c_out.at[c, s], sem).wait()

    return sc_kernel(z, sources, sinks,
                     jnp.arange(N_PAD, dtype=jnp.int32).reshape(N_PAD // K, K),
                     zrow_zeros, cnt_zeros)


def _tc_combine(keys, a0, a1, cw, w1t, w2t, b2):
    BLK = 1024

    def body(keys_b, a0_b, a1_b, cw_b, w1_b, w2_b, b_b, o_b):
        cnt = jnp.sum(cw_b[...], axis=0)[:, None]
        kk = jnp.dot(keys_b[...], w1_b[...],
                     preferred_element_type=jnp.float32)
        aa = jnp.dot(a0_b[...] + a1_b[...], w2_b[...],
                     preferred_element_type=jnp.float32)
        o_b[...] = (cnt * (kk + b_b[...]) + aa) / jnp.maximum(cnt, 1.0)

    row_spec = pl.BlockSpec((BLK, LATENT), lambda i: (i, 0))
    cnt_spec = pl.BlockSpec((NW, BLK), lambda i: (0, i))
    mat_spec = pl.BlockSpec((LATENT, LATENT), lambda i: (0, 0))
    return pl.pallas_call(
        body,
        grid=((N_NODES + BLK - 1) // BLK,),
        in_specs=[row_spec, row_spec, row_spec, cnt_spec,
                  mat_spec, mat_spec, pl.BlockSpec((1, INPUT), lambda i: (0, 0))],
        out_specs=pl.BlockSpec((BLK, INPUT), lambda i: (i, 0)),
        out_shape=jax.ShapeDtypeStruct((N_NODES, INPUT), jnp.float32),
    )(keys, a0, a1, cw, w1t, w2t, b2)


def kernel(z, keys, source_sink, W, b):
    zrow_zeros = jnp.zeros((K, LATENT), jnp.float32)
    cnt_zeros = jnp.zeros((N_PAD,), jnp.float32)
    a_part, c_part = _sc_segment_sum(z, source_sink[0], source_sink[1],
                                     zrow_zeros, cnt_zeros)
    w1t = W[:, :LATENT].T
    w2t = W[:, LATENT:].T
    b2 = b.reshape(1, INPUT)
    return _tc_combine(keys, a_part[0], a_part[1],
                       c_part.reshape(NW, N_PAD), w1t, w2t, b2)


# speedup vs baseline: 17.3517x; 1.1398x over previous
import dataclasses
import functools

import jax
import jax.numpy as jnp
from jax import lax
from jax.experimental import pallas as pl
from jax.experimental.pallas import tpu as pltpu
from jax.experimental.pallas import tpu_sc as plsc

N_NODES = 10000
N_EDGES = 320000
LATENT = 128
INPUT = 128

NC = 2
NS = 16
NW = NC * NS
K = 128
NBT = N_EDGES // K
NB2 = NBT // NW
NTAIL = NBT - NB2 * NW
N_PAD = 10240
RPW = N_PAD // NS
KI = 80
NI = RPW // KI


def _sc_segment_sum(z, sources, sinks, zrow_zeros, cnt_zeros):
    mesh = plsc.VectorSubcoreMesh(
        core_axis_name="c", subcore_axis_name="s",
        num_cores=NC, num_subcores=NS)

    cp = pltpu.CompilerParams()
    if "needs_layout_passes" in pltpu.CompilerParams.__dataclass_fields__:
        cp = dataclasses.replace(cp, needs_layout_passes=False)

    @functools.partial(
        pl.kernel,
        out_type=[
            jax.ShapeDtypeStruct((NC, N_PAD, LATENT), jnp.float32),
            jax.ShapeDtypeStruct((NC, NS, N_PAD), jnp.float32),
        ],
        compiler_params=cp,
        mesh=mesh,
        scratch_types=[
            pltpu.VMEM_SHARED((N_PAD, LATENT), jnp.float32),
            pltpu.VMEM((1, K), jnp.int32),
            pltpu.VMEM((1, K), jnp.int32),
            pltpu.VMEM((1, K), jnp.int32),
            pltpu.VMEM((1, K), jnp.int32),
            pltpu.VMEM((NI, KI), jnp.int32),
            pltpu.VMEM((K, LATENT), jnp.float32),
            pltpu.VMEM((K, LATENT), jnp.float32),
            pltpu.VMEM((N_PAD,), jnp.float32),
            pltpu.SemaphoreType.DMA,
            pltpu.SemaphoreType.DMA,
            pltpu.SemaphoreType.DMA,
            pltpu.SemaphoreType.DMA,
            pltpu.SemaphoreType.DMA,
            pltpu.SemaphoreType.DMA,
            pltpu.SemaphoreType.DMA,
        ],
    )
    def sc_kernel(z_hbm, src_hbm, snk_hbm, ar_hbm, zz_hbm, cz_hbm,
                  a_out, c_out, acc, sidx_a, tidx_a, sidx_b, tidx_b, iidx8,
                  rows_a, rows_b, cnt_v,
                  sem, is_a, is_b, gs_a, gs_b, ss_a, ss_b):
        c = lax.axis_index("c")
        s = lax.axis_index("s")
        wid = c * NS + s
        ones16 = jnp.ones((16,), jnp.float32)
        zrow = rows_a.at[pl.ds(0, KI)]

        def load_idx(b, sidx, tidx, isem):
            base = b * K
            pltpu.async_copy(src_hbm.at[pl.ds(base, K)], sidx.at[0], isem)
            pltpu.async_copy(snk_hbm.at[pl.ds(base, K)], tidx.at[0], isem)

        def wait_idx(sidx, tidx, isem):
            pltpu.make_async_copy(src_hbm.at[pl.ds(0, K)], sidx.at[0],
                                  isem).wait()
            pltpu.make_async_copy(snk_hbm.at[pl.ds(0, K)], tidx.at[0],
                                  isem).wait()

        def issue_gather(tidx, rows, gsem):
            pltpu.async_copy(z_hbm.at[tidx.at[0]], rows, gsem)

        def wait_gather(tidx, rows, gsem):
            pltpu.make_async_copy(z_hbm.at[tidx.at[0]], rows, gsem).wait()

        def issue_scat(rows, sidx, ssem):
            pltpu.async_copy(rows, acc.at[sidx.at[0]], ssem, add=True)

        def wait_scat(rows, sidx, ssem):
            pltpu.make_async_copy(rows, acc.at[sidx.at[0]], ssem).wait()

        def counts(sidx):
            @pl.loop(0, K // 16)
            def _(jj):
                idx16 = sidx[0, pl.ds(jj * 16, 16)]
                plsc.addupdate_scatter(cnt_v, [idx16], ones16)

        pltpu.sync_copy(zz_hbm, zrow)
        pltpu.sync_copy(cz_hbm, cnt_v)
        pltpu.sync_copy(ar_hbm.at[pl.ds(s * NI, NI)], iidx8)

        @pl.loop(0, NI)
        def _(j):
            pltpu.async_copy(zrow, acc.at[iidx8.at[j]], sem)

        b0 = wid * NB2
        load_idx(b0, sidx_a, tidx_a, is_a)
        load_idx(b0 + 1, sidx_b, tidx_b, is_b)

        @pl.loop(0, NI)
        def _(j):
            pltpu.make_async_copy(zrow, acc.at[iidx8.at[0]], sem).wait()

        plsc.subcore_barrier()

        wait_idx(sidx_a, tidx_a, is_a)
        issue_gather(tidx_a, rows_a, gs_a)

        @pl.loop(0, NB2 // 2)
        def _(j):
            e = 2 * j
            wait_gather(tidx_a, rows_a, gs_a)
            issue_scat(rows_a, sidx_a, ss_a)
            counts(sidx_a)
            wait_idx(sidx_b, tidx_b, is_b)
            issue_gather(tidx_b, rows_b, gs_b)
            wait_scat(rows_a, sidx_a, ss_a)

            @pl.when(e + 2 < NB2)
            def _():
                load_idx(b0 + e + 2, sidx_a, tidx_a, is_a)

            wait_gather(tidx_b, rows_b, gs_b)
            issue_scat(rows_b, sidx_b, ss_b)
            counts(sidx_b)

            @pl.when(e + 2 < NB2)
            def _():
                wait_idx(sidx_a, tidx_a, is_a)
                issue_gather(tidx_a, rows_a, gs_a)

            wait_scat(rows_b, sidx_b, ss_b)

            @pl.when(e + 3 < NB2)
            def _():
                load_idx(b0 + e + 3, sidx_b, tidx_b, is_b)

        @pl.when(wid < NTAIL)
        def _():
            load_idx(NB2 * NW + wid, sidx_a, tidx_a, is_a)
            wait_idx(sidx_a, tidx_a, is_a)
            pltpu.async_copy(z_hbm.at[tidx_a.at[0]], rows_a, gs_a).wait()
            issue_scat(rows_a, sidx_a, ss_a)
            counts(sidx_a)
            wait_scat(rows_a, sidx_a, ss_a)

        plsc.subcore_barrier()

        pltpu.async_copy(cnt_v, c_out.at[c, s], sem)
        ra = rows_a.at[pl.ds(0, KI)]
        rb = rows_b.at[pl.ds(0, KI)]

        def spg(j, rows, gsem):
            pltpu.async_copy(acc.at[iidx8.at[j]], rows, gsem)

        def wait_spg(rows, gsem):
            pltpu.make_async_copy(acc.at[iidx8.at[0]], rows, gsem).wait()

        def wout(j, rows, wsem):
            pltpu.async_copy(rows, a_out.at[c, pl.ds(s * RPW + j * KI, KI)],
                             wsem)

        def wait_wout(rows, wsem):
            pltpu.make_async_copy(rows, a_out.at[c, pl.ds(0, KI)],
                                  wsem).wait()

        spg(0, ra, gs_a)

        @pl.loop(0, NI // 2)
        def _(j):
            e = 2 * j
            wait_spg(ra, gs_a)
            wout(e, ra, ss_a)
            spg(e + 1, rb, gs_b)
            wait_spg(rb, gs_b)
            wout(e + 1, rb, ss_b)
            wait_wout(ra, ss_a)

            @pl.when(e + 2 < NI)
            def _():
                spg(e + 2, ra, gs_a)

            wait_wout(rb, ss_b)

        pltpu.make_async_copy(cnt_v, c_out.at[c, s], sem).wait()

    return sc_kernel(z, sources, sinks,
                     jnp.arange(N_PAD, dtype=jnp.int32).reshape(N_PAD // KI,
                                                                KI),
                     zrow_zeros, cnt_zeros)


def _tc_combine(keys, a0, a1, cw, w1t, w2t, b2):
    BLK = 1024

    def body(keys_b, a0_b, a1_b, cw_b, w1_b, w2_b, b_b, o_b):
        cnt = jnp.sum(cw_b[...], axis=0)[:, None]
        kk = jnp.dot(keys_b[...], w1_b[...],
                     preferred_element_type=jnp.float32)
        aa = jnp.dot(a0_b[...] + a1_b[...], w2_b[...],
                     preferred_element_type=jnp.float32)
        o_b[...] = (cnt * (kk + b_b[...]) + aa) / jnp.maximum(cnt, 1.0)

    row_spec = pl.BlockSpec((BLK, LATENT), lambda i: (i, 0))
    cnt_spec = pl.BlockSpec((NW, BLK), lambda i: (0, i))
    mat_spec = pl.BlockSpec((LATENT, LATENT), lambda i: (0, 0))
    return pl.pallas_call(
        body,
        grid=((N_NODES + BLK - 1) // BLK,),
        in_specs=[row_spec, row_spec, row_spec, cnt_spec,
                  mat_spec, mat_spec, pl.BlockSpec((1, INPUT), lambda i: (0, 0))],
        out_specs=pl.BlockSpec((BLK, INPUT), lambda i: (i, 0)),
        out_shape=jax.ShapeDtypeStruct((N_NODES, INPUT), jnp.float32),
    )(keys, a0, a1, cw, w1t, w2t, b2)


def kernel(z, keys, source_sink, W, b):
    zrow_zeros = jnp.zeros((KI, LATENT), jnp.float32)
    cnt_zeros = jnp.zeros((N_PAD,), jnp.float32)
    a_part, c_part = _sc_segment_sum(z, source_sink[0], source_sink[1],
                                     zrow_zeros, cnt_zeros)
    w1t = W[:, :LATENT].T
    w2t = W[:, LATENT:].T
    b2 = b.reshape(1, INPUT)
    return _tc_combine(keys, a_part[0], a_part[1],
                       c_part.reshape(NW, N_PAD), w1t, w2t, b2)


# speedup vs baseline: 17.8300x; 1.0276x over previous
import dataclasses
import functools

import jax
import jax.numpy as jnp
from jax import lax
from jax.experimental import pallas as pl
from jax.experimental.pallas import tpu as pltpu
from jax.experimental.pallas import tpu_sc as plsc

N_NODES = 10000
N_EDGES = 320000
LATENT = 128
INPUT = 128

NC = 2
NS = 16
NW = NC * NS
K = 128
NBT = N_EDGES // K
NB2 = NBT // NW
NTAIL = NBT - NB2 * NW
N_PAD = 10240
RPW = N_PAD // NS
KI = 80
NI = RPW // KI


def _sc_segment_sum(z, sources, sinks, zrow_zeros, cnt_zeros):
    mesh = plsc.VectorSubcoreMesh(
        core_axis_name="c", subcore_axis_name="s",
        num_cores=NC, num_subcores=NS)

    cp = pltpu.CompilerParams()
    if "needs_layout_passes" in pltpu.CompilerParams.__dataclass_fields__:
        cp = dataclasses.replace(cp, needs_layout_passes=False)

    @functools.partial(
        pl.kernel,
        out_type=[
            jax.ShapeDtypeStruct((NC, N_PAD, LATENT), jnp.float32),
            jax.ShapeDtypeStruct((NC, NS, N_PAD), jnp.float32),
        ],
        compiler_params=cp,
        mesh=mesh,
        scratch_types=[
            pltpu.VMEM_SHARED((N_PAD, LATENT), jnp.float32),
            pltpu.VMEM((1, K), jnp.int32),
            pltpu.VMEM((1, K), jnp.int32),
            pltpu.VMEM((1, K), jnp.int32),
            pltpu.VMEM((1, K), jnp.int32),
            pltpu.VMEM((NI, KI), jnp.int32),
            pltpu.VMEM((K, LATENT), jnp.float32),
            pltpu.VMEM((K, LATENT), jnp.float32),
            pltpu.VMEM((N_PAD,), jnp.float32),
            pltpu.SemaphoreType.DMA,
            pltpu.SemaphoreType.DMA,
            pltpu.SemaphoreType.DMA,
            pltpu.SemaphoreType.DMA,
            pltpu.SemaphoreType.DMA,
            pltpu.SemaphoreType.DMA,
            pltpu.SemaphoreType.DMA,
        ],
    )
    def sc_kernel(z_hbm, src_hbm, snk_hbm, ar_hbm, zz_hbm, cz_hbm,
                  a_out, c_out, acc, sidx_a, tidx_a, sidx_b, tidx_b, iidx8,
                  rows_a, rows_b, cnt_v,
                  sem, is_a, is_b, gs_a, gs_b, ss_a, ss_b):
        c = lax.axis_index("c")
        s = lax.axis_index("s")
        wid = c * NS + s
        ones16 = jnp.ones((16,), jnp.float32)
        zrow = rows_a.at[pl.ds(0, KI)]

        def load_idx(b, sidx, tidx, isem):
            base = b * K
            pltpu.async_copy(src_hbm.at[pl.ds(base, K)], sidx.at[0], isem)
            pltpu.async_copy(snk_hbm.at[pl.ds(base, K)], tidx.at[0], isem)

        def wait_idx(sidx, tidx, isem):
            pltpu.make_async_copy(src_hbm.at[pl.ds(0, K)], sidx.at[0],
                                  isem).wait()
            pltpu.make_async_copy(snk_hbm.at[pl.ds(0, K)], tidx.at[0],
                                  isem).wait()

        def issue_gather(tidx, rows, gsem):
            pltpu.async_copy(z_hbm.at[tidx.at[0]], rows, gsem)

        def wait_gather(tidx, rows, gsem):
            pltpu.make_async_copy(z_hbm.at[tidx.at[0]], rows, gsem).wait()

        def issue_scat(rows, sidx, ssem):
            pltpu.async_copy(rows, acc.at[sidx.at[0]], ssem, add=True)

        def wait_scat(rows, sidx, ssem):
            pltpu.make_async_copy(rows, acc.at[sidx.at[0]], ssem).wait()

        def counts(sidx):
            @pl.loop(0, K // 16)
            def _(jj):
                idx16 = sidx[0, pl.ds(jj * 16, 16)]
                plsc.addupdate_scatter(cnt_v, [idx16], ones16)

        pltpu.sync_copy(zz_hbm, zrow)
        pltpu.sync_copy(cz_hbm, cnt_v)
        pltpu.sync_copy(ar_hbm.at[pl.ds(s * NI, NI)], iidx8)

        @pl.loop(0, NI)
        def _(j):
            pltpu.async_copy(zrow, acc.at[iidx8.at[j]], sem)

        b0 = wid * NB2
        load_idx(b0, sidx_a, tidx_a, is_a)
        load_idx(b0 + 1, sidx_b, tidx_b, is_b)

        @pl.loop(0, NI)
        def _(j):
            pltpu.make_async_copy(zrow, acc.at[iidx8.at[0]], sem).wait()

        plsc.subcore_barrier()

        wait_idx(sidx_a, tidx_a, is_a)
        issue_gather(tidx_a, rows_a, gs_a)

        @pl.loop(0, NB2 // 2)
        def _(j):
            e = 2 * j
            wait_gather(tidx_a, rows_a, gs_a)
            issue_scat(rows_a, sidx_a, ss_a)
            counts(sidx_a)
            wait_idx(sidx_b, tidx_b, is_b)
            issue_gather(tidx_b, rows_b, gs_b)
            wait_scat(rows_a, sidx_a, ss_a)

            @pl.when(e + 2 < NB2)
            def _():
                load_idx(b0 + e + 2, sidx_a, tidx_a, is_a)

            wait_gather(tidx_b, rows_b, gs_b)
            issue_scat(rows_b, sidx_b, ss_b)
            counts(sidx_b)

            @pl.when(e + 2 < NB2)
            def _():
                wait_idx(sidx_a, tidx_a, is_a)
                issue_gather(tidx_a, rows_a, gs_a)

            wait_scat(rows_b, sidx_b, ss_b)

            @pl.when(e + 3 < NB2)
            def _():
                load_idx(b0 + e + 3, sidx_b, tidx_b, is_b)

        @pl.when(wid < NTAIL)
        def _():
            load_idx(NB2 * NW + wid, sidx_a, tidx_a, is_a)
            wait_idx(sidx_a, tidx_a, is_a)
            pltpu.async_copy(z_hbm.at[tidx_a.at[0]], rows_a, gs_a).wait()
            issue_scat(rows_a, sidx_a, ss_a)
            counts(sidx_a)
            wait_scat(rows_a, sidx_a, ss_a)

        plsc.subcore_barrier()

        pltpu.async_copy(cnt_v, c_out.at[c, s], sem)
        ra = rows_a.at[pl.ds(0, KI)]
        rb = rows_b.at[pl.ds(0, KI)]

        def spg(j, rows, gsem):
            pltpu.async_copy(acc.at[iidx8.at[j]], rows, gsem)

        def wait_spg(rows, gsem):
            pltpu.make_async_copy(acc.at[iidx8.at[0]], rows, gsem).wait()

        def wout(j, rows, wsem):
            pltpu.async_copy(rows, a_out.at[c, pl.ds(s * RPW + j * KI, KI)],
                             wsem)

        def wait_wout(rows, wsem):
            pltpu.make_async_copy(rows, a_out.at[c, pl.ds(0, KI)],
                                  wsem).wait()

        spg(0, ra, gs_a)

        @pl.loop(0, NI // 2)
        def _(j):
            e = 2 * j
            wait_spg(ra, gs_a)
            wout(e, ra, ss_a)
            spg(e + 1, rb, gs_b)
            wait_spg(rb, gs_b)
            wout(e + 1, rb, ss_b)
            wait_wout(ra, ss_a)

            @pl.when(e + 2 < NI)
            def _():
                spg(e + 2, ra, gs_a)

            wait_wout(rb, ss_b)

        pltpu.make_async_copy(cnt_v, c_out.at[c, s], sem).wait()

    return sc_kernel(z, sources, sinks,
                     jnp.arange(N_PAD, dtype=jnp.int32).reshape(N_PAD // KI,
                                                                KI),
                     zrow_zeros, cnt_zeros)


def _tc_keys_term(keys, w1t, b2):
    BLK = 1024

    def body(keys_b, w1_b, b_b, o_b):
        o_b[...] = jnp.dot(keys_b[...], w1_b[...],
                           preferred_element_type=jnp.float32) + b_b[...]

    return pl.pallas_call(
        body,
        grid=((N_NODES + BLK - 1) // BLK,),
        in_specs=[pl.BlockSpec((BLK, LATENT), lambda i: (i, 0)),
                  pl.BlockSpec((LATENT, LATENT), lambda i: (0, 0)),
                  pl.BlockSpec((1, INPUT), lambda i: (0, 0))],
        out_specs=pl.BlockSpec((BLK, INPUT), lambda i: (i, 0)),
        out_shape=jax.ShapeDtypeStruct((N_NODES, INPUT), jnp.float32),
    )(keys, w1t, b2)


def _tc_combine(kk, a_part, cw, w2t):
    BLK = 1024

    def body(kk_b, ap_b, cw_b, w2_b, o_b):
        cnt = jnp.sum(cw_b[...], axis=0)[:, None]
        aa = jnp.dot(ap_b[0] + ap_b[1], w2_b[...],
                     preferred_element_type=jnp.float32)
        o_b[...] = (cnt * kk_b[...] + aa) / jnp.maximum(cnt, 1.0)

    return pl.pallas_call(
        body,
        grid=((N_NODES + BLK - 1) // BLK,),
        in_specs=[pl.BlockSpec((BLK, INPUT), lambda i: (i, 0)),
                  pl.BlockSpec((NC, BLK, LATENT), lambda i: (0, i, 0)),
                  pl.BlockSpec((NW, BLK), lambda i: (0, i)),
                  pl.BlockSpec((LATENT, LATENT), lambda i: (0, 0))],
        out_specs=pl.BlockSpec((BLK, INPUT), lambda i: (i, 0)),
        out_shape=jax.ShapeDtypeStruct((N_NODES, INPUT), jnp.float32),
    )(kk, a_part, cw, w2t)


def kernel(z, keys, source_sink, W, b):
    zrow_zeros = jnp.zeros((KI, LATENT), jnp.float32)
    cnt_zeros = jnp.zeros((N_PAD,), jnp.float32)
    a_part, c_part = _sc_segment_sum(z, source_sink[0], source_sink[1],
                                     zrow_zeros, cnt_zeros)
    kk = _tc_keys_term(keys, W[:, :LATENT].T, b.reshape(1, INPUT))
    return _tc_combine(kk, a_part, c_part.reshape(NW, N_PAD),
                       W[:, LATENT:].T)


# speedup vs baseline: 18.3733x; 1.0305x over previous
import dataclasses
import functools

import jax
import jax.numpy as jnp
from jax import lax
from jax.experimental import pallas as pl
from jax.experimental.pallas import tpu as pltpu
from jax.experimental.pallas import tpu_sc as plsc

N_NODES = 10000
N_EDGES = 320000
LATENT = 128
INPUT = 128

NC = 2
NS = 16
NW = NC * NS
K = 64
NBT = N_EDGES // K
NB2 = NBT // NW
NBC = NB2 // 2
NTAIL = NBT - NB2 * NW
N_PAD = 10240
RPW = N_PAD // NS
KI = 64
NI = RPW // KI


def _sc_segment_sum(z, sources, sinks, zrow_zeros, cnt_zeros):
    mesh = plsc.VectorSubcoreMesh(
        core_axis_name="c", subcore_axis_name="s",
        num_cores=NC, num_subcores=NS)

    cp = pltpu.CompilerParams()
    if "needs_layout_passes" in pltpu.CompilerParams.__dataclass_fields__:
        cp = dataclasses.replace(cp, needs_layout_passes=False)

    @functools.partial(
        pl.kernel,
        out_type=[
            jax.ShapeDtypeStruct((NC, N_PAD, LATENT), jnp.float32),
            jax.ShapeDtypeStruct((NC, NS, N_PAD), jnp.float32),
        ],
        compiler_params=cp,
        mesh=mesh,
        scratch_types=[
            pltpu.VMEM_SHARED((N_PAD, LATENT), jnp.float32),
            pltpu.VMEM((1, K), jnp.int32),
            pltpu.VMEM((1, K), jnp.int32),
            pltpu.VMEM((1, K), jnp.int32),
            pltpu.VMEM((1, K), jnp.int32),
            pltpu.VMEM((1, K), jnp.int32),
            pltpu.VMEM((1, K), jnp.int32),
            pltpu.VMEM((1, K), jnp.int32),
            pltpu.VMEM((1, K), jnp.int32),
            pltpu.VMEM((NI, KI), jnp.int32),
            pltpu.VMEM((K, LATENT), jnp.float32),
            pltpu.VMEM((K, LATENT), jnp.float32),
            pltpu.VMEM((K, LATENT), jnp.float32),
            pltpu.VMEM((K, LATENT), jnp.float32),
            pltpu.VMEM((N_PAD,), jnp.float32),
            pltpu.SemaphoreType.DMA,
            pltpu.SemaphoreType.DMA,
            pltpu.SemaphoreType.DMA,
            pltpu.SemaphoreType.DMA,
            pltpu.SemaphoreType.DMA,
            pltpu.SemaphoreType.DMA,
            pltpu.SemaphoreType.DMA,
            pltpu.SemaphoreType.DMA,
            pltpu.SemaphoreType.DMA,
            pltpu.SemaphoreType.DMA,
            pltpu.SemaphoreType.DMA,
            pltpu.SemaphoreType.DMA,
            pltpu.SemaphoreType.DMA,
        ],
    )
    def sc_kernel(z_hbm, src_hbm, snk_hbm, ar_hbm, zz_hbm, cz_hbm,
                  a_out, c_out, acc,
                  sidx_a, tidx_a, sidx_b, tidx_b,
                  sidx_c, tidx_c, sidx_d, tidx_d, iidx8,
                  rows_a, rows_b, rows_c, rows_d, cnt_v,
                  sem, is_a, is_b, is_c, is_d,
                  gs_a, gs_b, gs_c, gs_d, ss_a, ss_b, ss_c, ss_d):
        c = lax.axis_index("c")
        s = lax.axis_index("s")
        wid = c * NS + s
        ones16 = jnp.ones((16,), jnp.float32)
        zrow = rows_a.at[pl.ds(0, KI)]

        def load_idx(b, sidx, tidx, isem):
            base = b * K
            pltpu.async_copy(src_hbm.at[pl.ds(base, K)], sidx.at[0], isem)
            pltpu.async_copy(snk_hbm.at[pl.ds(base, K)], tidx.at[0], isem)

        def wait_idx(sidx, tidx, isem):
            pltpu.make_async_copy(src_hbm.at[pl.ds(0, K)], sidx.at[0],
                                  isem).wait()
            pltpu.make_async_copy(snk_hbm.at[pl.ds(0, K)], tidx.at[0],
                                  isem).wait()

        def issue_gather(tidx, rows, gsem):
            pltpu.async_copy(z_hbm.at[tidx.at[0]], rows, gsem)

        def wait_gather(tidx, rows, gsem):
            pltpu.make_async_copy(z_hbm.at[tidx.at[0]], rows, gsem).wait()

        def issue_scat(rows, sidx, ssem):
            pltpu.async_copy(rows, acc.at[sidx.at[0]], ssem, add=True)

        def wait_scat(rows, sidx, ssem):
            pltpu.make_async_copy(rows, acc.at[sidx.at[0]], ssem).wait()

        def counts(sidx):
            @pl.loop(0, K // 16)
            def _(jj):
                idx16 = sidx[0, pl.ds(jj * 16, 16)]
                plsc.addupdate_scatter(cnt_v, [idx16], ones16)

        pltpu.sync_copy(zz_hbm, zrow)
        pltpu.sync_copy(cz_hbm, cnt_v)
        pltpu.sync_copy(ar_hbm.at[s], iidx8)

        @pl.loop(0, NI)
        def _(j):
            pltpu.async_copy(zrow, acc.at[iidx8.at[j]], sem)

        b1 = wid * NB2
        b2 = b1 + NBC
        load_idx(b1, sidx_a, tidx_a, is_a)
        load_idx(b2, sidx_c, tidx_c, is_c)
        load_idx(b1 + 1, sidx_b, tidx_b, is_b)
        load_idx(b2 + 1, sidx_d, tidx_d, is_d)

        @pl.loop(0, NI)
        def _(j):
            pltpu.make_async_copy(zrow, acc.at[iidx8.at[0]], sem).wait()

        plsc.subcore_barrier()

        wait_idx(sidx_a, tidx_a, is_a)
        issue_gather(tidx_a, rows_a, gs_a)
        wait_idx(sidx_c, tidx_c, is_c)
        issue_gather(tidx_c, rows_c, gs_c)

        @pl.loop(0, NBC // 2)
        def _(j):
            e = 2 * j
            wait_gather(tidx_a, rows_a, gs_a)
            issue_scat(rows_a, sidx_a, ss_a)
            wait_gather(tidx_c, rows_c, gs_c)
            issue_scat(rows_c, sidx_c, ss_c)
            counts(sidx_a)
            counts(sidx_c)
            wait_idx(sidx_b, tidx_b, is_b)
            issue_gather(tidx_b, rows_b, gs_b)
            wait_idx(sidx_d, tidx_d, is_d)
            issue_gather(tidx_d, rows_d, gs_d)
            wait_scat(rows_a, sidx_a, ss_a)
            wait_scat(rows_c, sidx_c, ss_c)

            @pl.when(e + 2 < NBC)
            def _():
                load_idx(b1 + e + 2, sidx_a, tidx_a, is_a)
                load_idx(b2 + e + 2, sidx_c, tidx_c, is_c)

            wait_gather(tidx_b, rows_b, gs_b)
            issue_scat(rows_b, sidx_b, ss_b)
            wait_gather(tidx_d, rows_d, gs_d)
            issue_scat(rows_d, sidx_d, ss_d)
            counts(sidx_b)
            counts(sidx_d)

            @pl.when(e + 2 < NBC)
            def _():
                wait_idx(sidx_a, tidx_a, is_a)
                issue_gather(tidx_a, rows_a, gs_a)
                wait_idx(sidx_c, tidx_c, is_c)
                issue_gather(tidx_c, rows_c, gs_c)

            wait_scat(rows_b, sidx_b, ss_b)
            wait_scat(rows_d, sidx_d, ss_d)

            @pl.when(e + 3 < NBC)
            def _():
                load_idx(b1 + e + 3, sidx_b, tidx_b, is_b)
                load_idx(b2 + e + 3, sidx_d, tidx_d, is_d)

        @pl.when(wid < NTAIL)
        def _():
            load_idx(NB2 * NW + wid, sidx_a, tidx_a, is_a)
            wait_idx(sidx_a, tidx_a, is_a)
            pltpu.async_copy(z_hbm.at[tidx_a.at[0]], rows_a, gs_a).wait()
            issue_scat(rows_a, sidx_a, ss_a)
            counts(sidx_a)
            wait_scat(rows_a, sidx_a, ss_a)

        plsc.subcore_barrier()

        pltpu.async_copy(cnt_v, c_out.at[c, s], sem)
        ra = rows_a.at[pl.ds(0, KI)]
        rb = rows_b.at[pl.ds(0, KI)]

        def spg(j, rows, gsem):
            pltpu.async_copy(acc.at[iidx8.at[j]], rows, gsem)

        def wait_spg(rows, gsem):
            pltpu.make_async_copy(acc.at[iidx8.at[0]], rows, gsem).wait()

        def wout(j, rows, wsem):
            pltpu.async_copy(rows, a_out.at[c, pl.ds(s * RPW + j * KI, KI)],
                             wsem)

        def wait_wout(rows, wsem):
            pltpu.make_async_copy(rows, a_out.at[c, pl.ds(0, KI)],
                                  wsem).wait()

        spg(0, ra, gs_a)

        @pl.loop(0, NI // 2)
        def _(j):
            e = 2 * j
            wait_spg(ra, gs_a)
            wout(e, ra, ss_a)
            spg(e + 1, rb, gs_b)
            wait_spg(rb, gs_b)
            wout(e + 1, rb, ss_b)
            wait_wout(ra, ss_a)

            @pl.when(e + 2 < NI)
            def _():
                spg(e + 2, ra, gs_a)

            wait_wout(rb, ss_b)

        pltpu.make_async_copy(cnt_v, c_out.at[c, s], sem).wait()

    return sc_kernel(z, sources, sinks,
                     jnp.arange(N_PAD, dtype=jnp.int32).reshape(NS, NI, KI),
                     zrow_zeros, cnt_zeros)


def _tc_keys_term(keys, w1t, b2):
    BLK = 1024

    def body(keys_b, w1_b, b_b, o_b):
        o_b[...] = jnp.dot(keys_b[...], w1_b[...],
                           preferred_element_type=jnp.float32) + b_b[...]

    return pl.pallas_call(
        body,
        grid=((N_NODES + BLK - 1) // BLK,),
        in_specs=[pl.BlockSpec((BLK, LATENT), lambda i: (i, 0)),
                  pl.BlockSpec((LATENT, LATENT), lambda i: (0, 0)),
                  pl.BlockSpec((1, INPUT), lambda i: (0, 0))],
        out_specs=pl.BlockSpec((BLK, INPUT), lambda i: (i, 0)),
        out_shape=jax.ShapeDtypeStruct((N_NODES, INPUT), jnp.float32),
    )(keys, w1t, b2)


def _tc_combine(kk, a_part, cw, w2t):
    BLK = 1024

    def body(kk_b, ap_b, cw_b, w2_b, o_b):
        cnt = jnp.sum(cw_b[...], axis=0)[:, None]
        aa = jnp.dot(ap_b[0] + ap_b[1], w2_b[...],
                     preferred_element_type=jnp.float32)
        o_b[...] = (cnt * kk_b[...] + aa) / jnp.maximum(cnt, 1.0)

    return pl.pallas_call(
        body,
        grid=((N_NODES + BLK - 1) // BLK,),
        in_specs=[pl.BlockSpec((BLK, INPUT), lambda i: (i, 0)),
                  pl.BlockSpec((NC, BLK, LATENT), lambda i: (0, i, 0)),
                  pl.BlockSpec((NW, BLK), lambda i: (0, i)),
                  pl.BlockSpec((LATENT, LATENT), lambda i: (0, 0))],
        out_specs=pl.BlockSpec((BLK, INPUT), lambda i: (i, 0)),
        out_shape=jax.ShapeDtypeStruct((N_NODES, INPUT), jnp.float32),
    )(kk, a_part, cw, w2t)


def kernel(z, keys, source_sink, W, b):
    zrow_zeros = jnp.zeros((KI, LATENT), jnp.float32)
    cnt_zeros = jnp.zeros((N_PAD,), jnp.float32)
    a_part, c_part = _sc_segment_sum(z, source_sink[0], source_sink[1],
                                     zrow_zeros, cnt_zeros)
    kk = _tc_keys_term(keys, W[:, :LATENT].T, b.reshape(1, INPUT))
    return _tc_combine(kk, a_part, c_part.reshape(NW, N_PAD),
                       W[:, LATENT:].T)
